# baseline (speedup 1.0000x reference)
from functools import partial

import jax
import jax.numpy as jnp
from jax.experimental import pallas as pl
from jax.experimental.pallas import tpu as pltpu


def _encoder_body(name_idx_ref, ingr_idx_ref,
                  tbl_ref, cal_ids_ref, cal_emb_ref, cal_w_ref, cal_b_ref,
                  ingr_w_ref, ingr_b_ref,
                  wif_ref, wib_ref, whf_ref, whb_ref,
                  bif_ref, bhf_ref, bib_ref, bhb_ref,
                  slab_ref, cal_out_ref, name_out_ref, ingr_out_ref,
                  x_scr, ingr_scr, *, T, H, NI, NT, Bt):
    De = tbl_ref.shape[2]
    H2 = 2 * H
    i = pl.program_id(0)
    bf16 = jnp.bfloat16
    f32 = jnp.float32

    def mm(a, b):
        return jnp.dot(a, b, preferred_element_type=f32)

    name_off = i * (T * Bt)
    UN = 32

    def gather_names(m, _):
        base = m * UN
        for u in range(UN):
            idx = name_idx_ref[name_off + base + u]
            x_scr[pl.ds(m * (UN // 8) + u // 8, 1), pl.ds(u % 8, 1), :] = (
                tbl_ref[pl.ds(idx, 1)])
        return 0

    jax.lax.fori_loop(0, (T * Bt) // UN, gather_names, 0)

    ingr_off = i * (Bt * NI * NT)
    chunk = (Bt * NI) // T

    def ingr_chunk(t):
        for us in range(chunk):
            s = t * chunk + us
            p = s * NT
            acc = tbl_ref[pl.ds(ingr_idx_ref[ingr_off + p], 1)]
            for k in range(1, NT):
                acc = acc + tbl_ref[pl.ds(ingr_idx_ref[ingr_off + p + k], 1)]
            ingr_scr[pl.ds(s // 8, 1), pl.ds(s % 8, 1), :] = acc

    n_cal = cal_emb_ref.shape[0]
    onehot = (cal_ids_ref[...] ==
              jax.lax.broadcasted_iota(jnp.int32, (Bt, 128), 1)
              )[:, :n_cal].astype(bf16)
    cal_x = mm(onehot, cal_emb_ref[...].astype(bf16))
    cal_enc = jnp.maximum(
        mm(cal_x.astype(bf16), cal_w_ref[...].astype(bf16)) + cal_b_ref[...],
        0.0)
    cal_out_ref[...] = cal_enc

    sc = jnp.concatenate([jnp.full((1, H2), 0.5, f32),
                          jnp.ones((1, H), f32)], axis=1)
    wif = (wif_ref[...] * sc).astype(bf16)
    wib = (wib_ref[...] * sc).astype(bf16)
    whf = (whf_ref[...] * sc).astype(bf16)
    whb = (whb_ref[...] * sc).astype(bf16)
    zH = jnp.zeros((1, H), f32)
    bpre_f = (bif_ref[...] +
              jnp.concatenate([bhf_ref[:, :H2], zH], axis=1)) * sc
    bpre_b = (bib_ref[...] +
              jnp.concatenate([bhb_ref[:, :H2], zH], axis=1)) * sc
    bhn_f = bhf_ref[0:1, H2:]
    bhn_b = bhb_ref[0:1, H2:]

    def gru_step(h, xs, wi, bpre, wh, bhn):
        gi = mm(xs, wi) + bpre
        gh = mm(h.astype(bf16), wh)
        rz = 0.5 * jnp.tanh(gi[:, :H2] + gh[:, :H2]) + 0.5
        n = jnp.tanh(gi[:, H2:] + rz[:, :H] * (gh[:, H2:] + bhn))
        z = rz[:, H:]
        return (1.0 - z) * n + z * h

    hf = jnp.zeros((Bt, H), f32)
    hb = jnp.zeros((Bt, H), f32)
    for t in range(T):
        b8 = Bt // 8
        xf = x_scr[t * b8:(t + 1) * b8].reshape(Bt, De).astype(bf16)
        xb = x_scr[(T - 1 - t) * b8:(T - t) * b8].reshape(Bt, De).astype(bf16)
        hf = gru_step(hf, xf, wif, bpre_f, whf, bhn_f)
        hb = gru_step(hb, xb, wib, bpre_b, whb, bhn_b)
        ingr_chunk(t)

    ingr_w = (ingr_w_ref[...] * (1.0 / NT)).astype(bf16)
    e = jnp.maximum(
        mm(ingr_scr[...].reshape(-1, De).astype(bf16), ingr_w) + ingr_b_ref[...],
        0.0)
    ingr_out_ref[...] = e
    pooled = e.reshape(Bt, NI, H).sum(axis=1) * (1.0 / NI)

    name_enc = jnp.concatenate([hf, hb], axis=1)
    name_out_ref[...] = name_enc
    slab_ref[...] = jnp.concatenate([cal_enc, name_enc, pooled], axis=1)


def kernel(vocab_emb, cal_emb, cal_w, cal_b, ingr_w, ingr_b,
           gru_wif, gru_whf, gru_bif, gru_bhf, gru_wib, gru_whb, gru_bib,
           gru_bhb, batch_calories, batch_names, batch_ingr):
    NI, NT = 8, 4
    B, T = batch_names.shape
    De = vocab_emb.shape[1]
    H = cal_w.shape[1]

    Bt = 256
    Bp = ((B + Bt - 1) // Bt) * Bt
    pad = Bp - B
    nblk = Bp // Bt

    names_t = batch_names.T
    cals = batch_calories
    ingrs = batch_ingr
    if pad:
        names_t = jnp.pad(names_t, ((0, 0), (0, pad)))
        cals = jnp.pad(cals, (0, pad))
        ingrs = jnp.pad(ingrs, ((0, pad), (0, 0)))
    name_idx = names_t.reshape(T, nblk, Bt).transpose(1, 0, 2).reshape(-1)
    ingr_idx = ingrs.reshape(-1)
    cal_ids = jnp.broadcast_to(cals[:, None], (Bp, 128)).astype(jnp.int32)

    tbl = vocab_emb.reshape(vocab_emb.shape[0], 1, De)

    body = partial(_encoder_body, T=T, H=H, NI=NI, NT=NT, Bt=Bt)
    slab, cal_enc, name_enc, ingr_out = pl.pallas_call(
        body,
        out_shape=(jax.ShapeDtypeStruct((Bp, 4 * H), jnp.float32),
                   jax.ShapeDtypeStruct((Bp, H), jnp.float32),
                   jax.ShapeDtypeStruct((Bp, 2 * H), jnp.float32),
                   jax.ShapeDtypeStruct((Bp * NI, H), jnp.float32)),
        grid_spec=pltpu.PrefetchScalarGridSpec(
            num_scalar_prefetch=2, grid=(nblk,),
            in_specs=[
                pl.BlockSpec((tbl.shape[0], 1, De), lambda i, *_: (0, 0, 0)),
                pl.BlockSpec((Bt, 128), lambda i, *_: (i, 0)),
                pl.BlockSpec(cal_emb.shape, lambda i, *_: (0, 0)),
                pl.BlockSpec((cal_emb.shape[1], H), lambda i, *_: (0, 0)),
                pl.BlockSpec((1, H), lambda i, *_: (0, 0)),
                pl.BlockSpec((De, H), lambda i, *_: (0, 0)),
                pl.BlockSpec((1, H), lambda i, *_: (0, 0)),
                pl.BlockSpec((De, 3 * H), lambda i, *_: (0, 0)),
                pl.BlockSpec((De, 3 * H), lambda i, *_: (0, 0)),
                pl.BlockSpec((H, 3 * H), lambda i, *_: (0, 0)),
                pl.BlockSpec((H, 3 * H), lambda i, *_: (0, 0)),
                pl.BlockSpec((1, 3 * H), lambda i, *_: (0, 0)),
                pl.BlockSpec((1, 3 * H), lambda i, *_: (0, 0)),
                pl.BlockSpec((1, 3 * H), lambda i, *_: (0, 0)),
                pl.BlockSpec((1, 3 * H), lambda i, *_: (0, 0)),
            ],
            out_specs=(pl.BlockSpec((Bt, 4 * H), lambda i, *_: (i, 0)),
                       pl.BlockSpec((Bt, H), lambda i, *_: (i, 0)),
                       pl.BlockSpec((Bt, 2 * H), lambda i, *_: (i, 0)),
                       pl.BlockSpec((Bt * NI, H), lambda i, *_: (i, 0))),
            scratch_shapes=[pltpu.VMEM((T * Bt // 8, 8, De), jnp.float32),
                            pltpu.VMEM((Bt * NI // 8, 8, De), jnp.float32)]),
        compiler_params=pltpu.CompilerParams(
            dimension_semantics=("parallel",),
            disable_bounds_checks=True,
            vmem_limit_bytes=64 * 1024 * 1024),
    )(name_idx, ingr_idx,
      tbl, cal_ids, cal_emb, cal_w, cal_b, ingr_w, ingr_b,
      gru_wif, gru_wib, gru_whf, gru_whb,
      gru_bif, gru_bhf, gru_bib, gru_bhb)

    recipe_encoding = slab[:B][None]
    calorie_level_encoding = cal_enc[:B]
    recipe_name_encoding = name_enc[:B][None]
    ingr_encodings = ingr_out[:B * NI].reshape(B, NI, H)
    return (recipe_encoding, calorie_level_encoding, recipe_name_encoding,
            ingr_encodings)


# speedup vs baseline: 1.0065x; 1.0065x over previous
from functools import partial

import jax
import jax.numpy as jnp
from jax.experimental import pallas as pl
from jax.experimental.pallas import tpu as pltpu


def _encoder_body(name_idx_ref, ingr_idx_ref,
                  tbl_ref, cal_ids_ref, cal_emb_ref, cal_w_ref, cal_b_ref,
                  ingr_w_ref, ingr_b_ref,
                  wif_ref, wib_ref, whf_ref, whb_ref,
                  bif_ref, bhf_ref, bib_ref, bhb_ref,
                  slab_ref, cal_out_ref, name_out_ref, ingr_out_ref,
                  x_scr, ingr_scr, *, T, H, NI, NT, Bt):
    De = tbl_ref.shape[2]
    H2 = 2 * H
    i = pl.program_id(0)
    bf16 = jnp.bfloat16
    f32 = jnp.float32

    def mm(a, b):
        return jnp.dot(a, b, preferred_element_type=f32)

    name_off = i * (T * Bt)
    UN = 32

    def gather_names(m, _):
        base = m * UN
        for u in range(UN):
            idx = name_idx_ref[name_off + base + u]
            x_scr[pl.ds(m * (UN // 8) + u // 8, 1), pl.ds(u % 8, 1), :] = (
                tbl_ref[pl.ds(idx, 1)])
        return 0

    jax.lax.fori_loop(0, (T * Bt) // UN, gather_names, 0)

    ingr_off = i * (Bt * NI * NT)
    chunk = (Bt * NI) // T

    def ingr_chunk(t):
        for us in range(chunk):
            s = t * chunk + us
            p = s * NT
            acc = tbl_ref[pl.ds(ingr_idx_ref[ingr_off + p], 1)]
            for k in range(1, NT):
                acc = acc + tbl_ref[pl.ds(ingr_idx_ref[ingr_off + p + k], 1)]
            ingr_scr[pl.ds(s // 8, 1), pl.ds(s % 8, 1), :] = acc

    n_cal = cal_emb_ref.shape[0]
    onehot = (cal_ids_ref[...] ==
              jax.lax.broadcasted_iota(jnp.int32, (Bt, 128), 1)
              )[:, :n_cal].astype(bf16)
    cal_x = mm(onehot, cal_emb_ref[...].astype(bf16))
    cal_enc = jnp.maximum(
        mm(cal_x.astype(bf16), cal_w_ref[...].astype(bf16)) + cal_b_ref[...],
        0.0)
    cal_out_ref[...] = cal_enc

    wif = wif_ref[...].astype(bf16)
    wib = wib_ref[...].astype(bf16)
    whf = whf_ref[...].astype(bf16)
    whb = whb_ref[...].astype(bf16)
    zH = jnp.zeros((1, H), f32)
    bpre_f = bif_ref[...] + jnp.concatenate([bhf_ref[:, :H2], zH], axis=1)
    bpre_b = bib_ref[...] + jnp.concatenate([bhb_ref[:, :H2], zH], axis=1)
    bhn_f = bhf_ref[0:1, H2:]
    bhn_b = bhb_ref[0:1, H2:]

    def gru_step(h, xs, wi, bpre, wh, bhn):
        gi = mm(xs, wi) + bpre
        gh = mm(h.astype(bf16), wh)
        rz = jax.nn.sigmoid(gi[:, :H2] + gh[:, :H2])
        n = jnp.tanh(gi[:, H2:] + rz[:, :H] * (gh[:, H2:] + bhn))
        z = rz[:, H:]
        return (1.0 - z) * n + z * h

    hf = jnp.zeros((Bt, H), f32)
    hb = jnp.zeros((Bt, H), f32)
    for t in range(T):
        b8 = Bt // 8
        xf = x_scr[t * b8:(t + 1) * b8].reshape(Bt, De).astype(bf16)
        xb = x_scr[(T - 1 - t) * b8:(T - t) * b8].reshape(Bt, De).astype(bf16)
        hf = gru_step(hf, xf, wif, bpre_f, whf, bhn_f)
        hb = gru_step(hb, xb, wib, bpre_b, whb, bhn_b)
        ingr_chunk(t)

    ingr_w = (ingr_w_ref[...] * (1.0 / NT)).astype(bf16)
    e = jnp.maximum(
        mm(ingr_scr[...].reshape(-1, De).astype(bf16), ingr_w) + ingr_b_ref[...],
        0.0)
    ingr_out_ref[...] = e
    pooled = e.reshape(Bt, NI, H).sum(axis=1) * (1.0 / NI)

    name_enc = jnp.concatenate([hf, hb], axis=1)
    name_out_ref[...] = name_enc
    slab_ref[...] = jnp.concatenate([cal_enc, name_enc, pooled], axis=1)


def kernel(vocab_emb, cal_emb, cal_w, cal_b, ingr_w, ingr_b,
           gru_wif, gru_whf, gru_bif, gru_bhf, gru_wib, gru_whb, gru_bib,
           gru_bhb, batch_calories, batch_names, batch_ingr):
    NI, NT = 8, 4
    B, T = batch_names.shape
    De = vocab_emb.shape[1]
    H = cal_w.shape[1]

    Bt = 256
    Bp = ((B + Bt - 1) // Bt) * Bt
    pad = Bp - B
    nblk = Bp // Bt

    names_t = batch_names.T
    cals = batch_calories
    ingrs = batch_ingr
    if pad:
        names_t = jnp.pad(names_t, ((0, 0), (0, pad)))
        cals = jnp.pad(cals, (0, pad))
        ingrs = jnp.pad(ingrs, ((0, pad), (0, 0)))
    name_idx = names_t.reshape(T, nblk, Bt).transpose(1, 0, 2).reshape(-1)
    ingr_idx = ingrs.reshape(-1)
    cal_ids = jnp.broadcast_to(cals[:, None], (Bp, 128)).astype(jnp.int32)

    tbl = vocab_emb.reshape(vocab_emb.shape[0], 1, De)

    body = partial(_encoder_body, T=T, H=H, NI=NI, NT=NT, Bt=Bt)
    slab, cal_enc, name_enc, ingr_out = pl.pallas_call(
        body,
        out_shape=(jax.ShapeDtypeStruct((Bp, 4 * H), jnp.float32),
                   jax.ShapeDtypeStruct((Bp, H), jnp.float32),
                   jax.ShapeDtypeStruct((Bp, 2 * H), jnp.float32),
                   jax.ShapeDtypeStruct((Bp * NI, H), jnp.float32)),
        grid_spec=pltpu.PrefetchScalarGridSpec(
            num_scalar_prefetch=2, grid=(nblk,),
            in_specs=[
                pl.BlockSpec((tbl.shape[0], 1, De), lambda i, *_: (0, 0, 0)),
                pl.BlockSpec((Bt, 128), lambda i, *_: (i, 0)),
                pl.BlockSpec(cal_emb.shape, lambda i, *_: (0, 0)),
                pl.BlockSpec((cal_emb.shape[1], H), lambda i, *_: (0, 0)),
                pl.BlockSpec((1, H), lambda i, *_: (0, 0)),
                pl.BlockSpec((De, H), lambda i, *_: (0, 0)),
                pl.BlockSpec((1, H), lambda i, *_: (0, 0)),
                pl.BlockSpec((De, 3 * H), lambda i, *_: (0, 0)),
                pl.BlockSpec((De, 3 * H), lambda i, *_: (0, 0)),
                pl.BlockSpec((H, 3 * H), lambda i, *_: (0, 0)),
                pl.BlockSpec((H, 3 * H), lambda i, *_: (0, 0)),
                pl.BlockSpec((1, 3 * H), lambda i, *_: (0, 0)),
                pl.BlockSpec((1, 3 * H), lambda i, *_: (0, 0)),
                pl.BlockSpec((1, 3 * H), lambda i, *_: (0, 0)),
                pl.BlockSpec((1, 3 * H), lambda i, *_: (0, 0)),
            ],
            out_specs=(pl.BlockSpec((Bt, 4 * H), lambda i, *_: (i, 0)),
                       pl.BlockSpec((Bt, H), lambda i, *_: (i, 0)),
                       pl.BlockSpec((Bt, 2 * H), lambda i, *_: (i, 0)),
                       pl.BlockSpec((Bt * NI, H), lambda i, *_: (i, 0))),
            scratch_shapes=[pltpu.VMEM((T * Bt // 8, 8, De), jnp.float32),
                            pltpu.VMEM((Bt * NI // 8, 8, De), jnp.float32)]),
        compiler_params=pltpu.CompilerParams(
            dimension_semantics=("parallel",),
            disable_bounds_checks=True,
            vmem_limit_bytes=64 * 1024 * 1024),
    )(name_idx, ingr_idx,
      tbl, cal_ids, cal_emb, cal_w, cal_b, ingr_w, ingr_b,
      gru_wif, gru_wib, gru_whf, gru_whb,
      gru_bif, gru_bhf, gru_bib, gru_bhb)

    recipe_encoding = slab[:B][None]
    calorie_level_encoding = cal_enc[:B]
    recipe_name_encoding = name_enc[:B][None]
    ingr_encodings = ingr_out[:B * NI].reshape(B, NI, H)
    return (recipe_encoding, calorie_level_encoding, recipe_name_encoding,
            ingr_encodings)
